# Initial kernel scaffold; baseline (speedup 1.0000x reference)
import jax
import jax.numpy as jnp
from jax.experimental import pallas as pl


def kernel(x, atten, W):
    raise NotImplementedError("write your pallas kernel here")


# speedup vs baseline: 1.1198x; 1.1198x over previous
import jax
import jax.numpy as jnp
from jax.experimental import pallas as pl
from jax.experimental.pallas import tpu as pltpu

_HEADS = 12


def _reduce_body(a_ref, out_ref):
    m = pl.program_id(0)
    r = pl.program_id(1)
    nr = pl.num_programs(1)

    @pl.when(jnp.logical_and(m % _HEADS == 0, r == 0))
    def _():
        out_ref[...] = jnp.zeros_like(out_ref)

    data = a_ref[0]
    R, S = data.shape
    rows = jax.lax.broadcasted_iota(jnp.int32, (R, S), 0) + r * R
    cols = jax.lax.broadcasted_iota(jnp.int32, (R, S), 1)
    contrib = jnp.where(rows == cols, 0.0, data)
    out_ref[0] += jnp.sum(contrib, axis=0, keepdims=True)


def _rank_gate_body(arow_ref, acol_ref, w_ref, x_ref, out_ref):
    sb = pl.program_id(1)
    a_row = arow_ref[0]
    a_col = acol_ref[...]
    SB = a_col.shape[0]
    S = a_row.shape[1]
    s_glob = jax.lax.broadcasted_iota(jnp.int32, (SB, S), 0) + sb * SB
    j_glob = jax.lax.broadcasted_iota(jnp.int32, (SB, S), 1)
    gt = a_row > a_col
    tie = jnp.logical_and(a_row == a_col, j_glob < s_glob)
    cmp = jnp.where(jnp.logical_or(gt, tie), 1.0, 0.0)
    rank = jnp.sum(cmp, axis=1, keepdims=True).astype(jnp.int32)
    onehot = jnp.where(j_glob == rank, 1.0, 0.0)
    gate = jnp.sum(onehot * w_ref[...], axis=1, keepdims=True)
    out_ref[0] = x_ref[0] * gate


def kernel(x, atten, W):
    B, S, D = x.shape
    BH = atten.shape[0]
    R = 256
    SB = 256
    nr = S // R
    nsb = S // SB

    attended = pl.pallas_call(
        _reduce_body,
        grid=(BH, nr),
        in_specs=[pl.BlockSpec((1, R, S), lambda m, r: (m, r, 0))],
        out_specs=pl.BlockSpec((1, 1, S), lambda m, r: (m // _HEADS, 0, 0)),
        out_shape=jax.ShapeDtypeStruct((B, 1, S), jnp.float32),
    )(atten)

    a_col = attended.reshape(B * S, 1)
    w_row = W.reshape(1, S)

    out = pl.pallas_call(
        _rank_gate_body,
        grid=(B, nsb),
        in_specs=[
            pl.BlockSpec((1, 1, S), lambda b, s: (b, 0, 0)),
            pl.BlockSpec((SB, 1), lambda b, s, _n=nsb: (b * _n + s, 0)),
            pl.BlockSpec((1, S), lambda b, s: (0, 0)),
            pl.BlockSpec((1, SB, D), lambda b, s: (b, s, 0)),
        ],
        out_specs=pl.BlockSpec((1, SB, D), lambda b, s: (b, s, 0)),
        out_shape=jax.ShapeDtypeStruct((B, S, D), jnp.float32),
    )(attended, a_col, w_row, x)
    return out


# speedup vs baseline: 1.4737x; 1.3161x over previous
import jax
import jax.numpy as jnp
from jax.experimental import pallas as pl
from jax.experimental.pallas import tpu as pltpu

_HEADS = 12


def _reduce_body(a_ref, out_ref):
    m = pl.program_id(0)
    r = pl.program_id(1)
    nr = pl.num_programs(1)

    @pl.when(jnp.logical_and(m % _HEADS == 0, r == 0))
    def _():
        out_ref[...] = jnp.zeros_like(out_ref)

    data = a_ref[0]
    R, S = data.shape
    rows = jax.lax.broadcasted_iota(jnp.int32, (R, S), 0) + r * R
    cols = jax.lax.broadcasted_iota(jnp.int32, (R, S), 1)
    contrib = jnp.where(rows == cols, 0.0, data)
    out_ref[0] += jnp.sum(contrib, axis=0, keepdims=True)


def _rank_gate_body(arow_ref, acol_ref, w_ref, x_ref, out_ref):
    sb = pl.program_id(1)
    a_row = arow_ref[0]
    a_col = acol_ref[...]
    SB = a_col.shape[0]
    S = a_row.shape[1]
    s_glob = jax.lax.broadcasted_iota(jnp.int32, (SB, S), 0) + sb * SB
    j_glob = jax.lax.broadcasted_iota(jnp.int32, (SB, S), 1)
    gt = a_row > a_col
    tie = jnp.logical_and(a_row == a_col, j_glob < s_glob)
    cmp = jnp.where(jnp.logical_or(gt, tie), 1.0, 0.0)
    rank = jnp.sum(cmp, axis=1, keepdims=True).astype(jnp.int32)
    onehot = jnp.where(j_glob == rank, 1.0, 0.0)
    gate = jnp.sum(onehot * w_ref[...], axis=1, keepdims=True)
    out_ref[0] = x_ref[0] * gate


def kernel(x, atten, W):
    B, S, D = x.shape
    BH = atten.shape[0]
    R = 512
    SB = 256
    nr = S // R
    nsb = S // SB

    attended = pl.pallas_call(
        _reduce_body,
        grid=(BH, nr),
        in_specs=[pl.BlockSpec((1, R, S), lambda m, r: (m, r, 0))],
        out_specs=pl.BlockSpec((1, 1, S), lambda m, r: (m // _HEADS, 0, 0)),
        out_shape=jax.ShapeDtypeStruct((B, 1, S), jnp.float32),
    )(atten)

    a_col = attended.reshape(B * S, 1)
    w_row = W.reshape(1, S)

    out = pl.pallas_call(
        _rank_gate_body,
        grid=(B, nsb),
        in_specs=[
            pl.BlockSpec((1, 1, S), lambda b, s: (b, 0, 0)),
            pl.BlockSpec((SB, 1), lambda b, s, _n=nsb: (b * _n + s, 0)),
            pl.BlockSpec((1, S), lambda b, s: (0, 0)),
            pl.BlockSpec((1, SB, D), lambda b, s: (b, s, 0)),
        ],
        out_specs=pl.BlockSpec((1, SB, D), lambda b, s: (b, s, 0)),
        out_shape=jax.ShapeDtypeStruct((B, S, D), jnp.float32),
    )(attended, a_col, w_row, x)
    return out


# speedup vs baseline: 1.6664x; 1.1307x over previous
import jax
import jax.numpy as jnp
from jax.experimental import pallas as pl
from jax.experimental.pallas import tpu as pltpu

_HEADS = 12


def _reduce_body(a_ref, out_ref):
    m = pl.program_id(0)
    r = pl.program_id(1)
    nr = pl.num_programs(1)

    @pl.when(jnp.logical_and(m % _HEADS == 0, r == 0))
    def _():
        out_ref[...] = jnp.zeros_like(out_ref)

    data = a_ref[0]
    R, S = data.shape
    rows = jax.lax.broadcasted_iota(jnp.int32, (R, S), 0) + r * R
    cols = jax.lax.broadcasted_iota(jnp.int32, (R, S), 1)
    contrib = jnp.where(rows == cols, 0.0, data)
    out_ref[0] += jnp.sum(contrib, axis=0, keepdims=True)


def _rank_gate_body(arow_ref, acol_ref, w_ref, x_ref, out_ref):
    sb = pl.program_id(1)
    a_row = arow_ref[0]
    a_col = acol_ref[...]
    SB = a_col.shape[0]
    S = a_row.shape[1]
    s_glob = jax.lax.broadcasted_iota(jnp.int32, (SB, S), 0) + sb * SB
    j_glob = jax.lax.broadcasted_iota(jnp.int32, (SB, S), 1)
    gt = a_row > a_col
    tie = jnp.logical_and(a_row == a_col, j_glob < s_glob)
    cmp = jnp.where(jnp.logical_or(gt, tie), 1.0, 0.0)
    rank = jnp.sum(cmp, axis=1, keepdims=True).astype(jnp.int32)
    onehot = jnp.where(j_glob == rank, 1.0, 0.0)
    gate = jnp.sum(onehot * w_ref[...], axis=1, keepdims=True)
    out_ref[0] = x_ref[0] * gate


def kernel(x, atten, W):
    B, S, D = x.shape
    BH = atten.shape[0]
    R = 1024
    SB = 256
    nr = S // R
    nsb = S // SB

    attended = pl.pallas_call(
        _reduce_body,
        grid=(BH, nr),
        in_specs=[pl.BlockSpec((1, R, S), lambda m, r: (m, r, 0))],
        out_specs=pl.BlockSpec((1, 1, S), lambda m, r: (m // _HEADS, 0, 0)),
        out_shape=jax.ShapeDtypeStruct((B, 1, S), jnp.float32),
    )(atten)

    a_col = attended.reshape(B * S, 1)
    w_row = W.reshape(1, S)

    out = pl.pallas_call(
        _rank_gate_body,
        grid=(B, nsb),
        in_specs=[
            pl.BlockSpec((1, 1, S), lambda b, s: (b, 0, 0)),
            pl.BlockSpec((SB, 1), lambda b, s, _n=nsb: (b * _n + s, 0)),
            pl.BlockSpec((1, S), lambda b, s: (0, 0)),
            pl.BlockSpec((1, SB, D), lambda b, s: (b, s, 0)),
        ],
        out_specs=pl.BlockSpec((1, SB, D), lambda b, s: (b, s, 0)),
        out_shape=jax.ShapeDtypeStruct((B, S, D), jnp.float32),
    )(attended, a_col, w_row, x)
    return out


# speedup vs baseline: 1.6891x; 1.0136x over previous
import jax
import jax.numpy as jnp
from jax.experimental import pallas as pl
from jax.experimental.pallas import tpu as pltpu

_HEADS = 12


def _reduce_body(a_ref, out_ref):
    m = pl.program_id(0)
    r = pl.program_id(1)
    nr = pl.num_programs(1)

    @pl.when(jnp.logical_and(m % _HEADS == 0, r == 0))
    def _():
        out_ref[...] = jnp.zeros_like(out_ref)

    data = a_ref[0]
    R, S = data.shape
    rows = jax.lax.broadcasted_iota(jnp.int32, (R, S), 0) + r * R
    cols = jax.lax.broadcasted_iota(jnp.int32, (R, S), 1)
    contrib = jnp.where(rows == cols, 0.0, data)
    out_ref[0] += jnp.sum(contrib, axis=0, keepdims=True)


def _rank_gate_body(arow_ref, acol_ref, w_ref, x_ref, out_ref):
    sb = pl.program_id(1)
    a_row = arow_ref[0]
    a_col = acol_ref[...]
    SB = a_col.shape[0]
    S = a_row.shape[1]
    s_glob = jax.lax.broadcasted_iota(jnp.int32, (SB, S), 0) + sb * SB
    j_glob = jax.lax.broadcasted_iota(jnp.int32, (SB, S), 1)
    gt = a_row > a_col
    tie = jnp.logical_and(a_row == a_col, j_glob < s_glob)
    cmp = jnp.where(jnp.logical_or(gt, tie), 1.0, 0.0)
    rank = jnp.sum(cmp, axis=1, keepdims=True).astype(jnp.int32)
    onehot = jnp.where(j_glob == rank, 1.0, 0.0)
    gate = jnp.sum(onehot * w_ref[...], axis=1, keepdims=True)
    out_ref[0] = x_ref[0] * gate


def kernel(x, atten, W):
    B, S, D = x.shape
    BH = atten.shape[0]
    R = 2048
    SB = 256
    nr = S // R
    nsb = S // SB

    attended = pl.pallas_call(
        _reduce_body,
        grid=(BH, nr),
        in_specs=[pl.BlockSpec((1, R, S), lambda m, r: (m, r, 0))],
        out_specs=pl.BlockSpec((1, 1, S), lambda m, r: (m // _HEADS, 0, 0)),
        out_shape=jax.ShapeDtypeStruct((B, 1, S), jnp.float32),
    )(atten)

    a_col = attended.reshape(B * S, 1)
    w_row = W.reshape(1, S)

    out = pl.pallas_call(
        _rank_gate_body,
        grid=(B, nsb),
        in_specs=[
            pl.BlockSpec((1, 1, S), lambda b, s: (b, 0, 0)),
            pl.BlockSpec((SB, 1), lambda b, s, _n=nsb: (b * _n + s, 0)),
            pl.BlockSpec((1, S), lambda b, s: (0, 0)),
            pl.BlockSpec((1, SB, D), lambda b, s: (b, s, 0)),
        ],
        out_specs=pl.BlockSpec((1, SB, D), lambda b, s: (b, s, 0)),
        out_shape=jax.ShapeDtypeStruct((B, S, D), jnp.float32),
    )(attended, a_col, w_row, x)
    return out
